# baseline (speedup 1.0000x reference)
import functools

import jax
import jax.numpy as jnp
from jax import lax
from jax.experimental import pallas as pl
from jax.experimental.pallas import tpu as pltpu
from jax.experimental.pallas import tpu_sc as plsc

BATCH = 16384
DIM = 64
NUM_CORES = 2
NUM_SUBCORES = 16
LANES = 16
NW = NUM_CORES * NUM_SUBCORES
BPW = BATCH // NW
BLK = 128
NBLK = BPW // BLK
NBAND = 8
GRP = 16
NGRP = BLK // GRP
NSLOT = 96
QD = 4


def _extract_body(su_hbm, si_hbm, eu_hbm, ei_hbm, gu_hbm, gi_hbm,
                  idxv, ring, rbuf, sem):
    wid = lax.axis_index("s") * NUM_CORES + lax.axis_index("c")
    base = wid * BPW
    lane_iota = lax.iota(jnp.int32, LANES)
    perm = lax.rem(lane_iota + 15, jnp.full((LANES,), 16, jnp.int32))
    lane0_i = (lane_iota == 0).astype(jnp.int32)
    NIT = NBLK * NBAND * NGRP

    for idx_hbm, tbl_hbm, out_hbm in ((su_hbm, eu_hbm, gu_hbm),
                                      (si_hbm, ei_hbm, gi_hbm)):
        pltpu.sync_copy(idx_hbm.at[pl.ds(base, BPW)], idxv)

        def decode(it):
            blk = it // (NBAND * NGRP)
            r = lax.rem(it, jnp.int32(NBAND * NGRP))
            band = r // NGRP
            g = lax.rem(r, jnp.int32(NGRP))
            return blk, band, g

        def ginfo_fire(it, S, active, tbl_hbm=tbl_hbm):
            blk, band, g = decode(it)
            off = blk * BLK + g * GRP
            iv = idxv[pl.ds(off, GRP)]
            tv = iv // 128
            lanes = lax.rem(iv, jnp.full((GRP,), 128, jnp.int32))
            shifted = tv.at[perm].get(mode="promise_in_bounds")
            poff = jnp.maximum(off - GRP, 0)
            prev = idxv[pl.ds(poff, GRP)] // 128
            pshift = prev.at[perm].get(mode="promise_in_bounds")
            shifted = jnp.where(lane_iota == 0, pshift, shifted)
            inew = (tv != shifted).astype(jnp.int32)
            force0 = (g == 0).astype(jnp.int32)
            inew = jnp.maximum(inew, lane0_i * force0)
            inew = inew * active
            slots = S + jnp.cumsum(inew) - 1
            brow = pl.multiple_of(band * NBAND, NBAND)
            for t in range(GRP):
                @pl.when(inew[t] == 1)
                def _(t=t):
                    tcol = pl.multiple_of(tv[t] * 128, 128)
                    pltpu.async_copy(
                        tbl_hbm.at[pl.ds(brow, NBAND), pl.ds(tcol, 128)],
                        ring.at[lax.rem(slots[t], jnp.int32(NSLOT))], sem)
            return lanes, slots, slots[GRP - 1] + 1

        def drain(n, tbl_hbm=tbl_hbm):
            def w(i, c):
                pltpu.make_async_copy(
                    tbl_hbm.at[pl.ds(0, NBAND), pl.ds(0, 128)],
                    ring.at[0], sem).wait()
                return c
            lax.fori_loop(0, n, w, 0)

        def extract(it, lanes, slots):
            blk, band, g = decode(it)
            smod = lax.rem(slots, jnp.full((GRP,), NSLOT, jnp.int32))
            rows = g * GRP + lane_iota
            for f in range(NBAND):
                vals = plsc.load_gather(
                    ring, [smod, jnp.full((GRP,), f, jnp.int32), lanes])
                plsc.store_scatter(
                    rbuf, [rows, band * NBAND + jnp.full((GRP,), f, jnp.int32)],
                    vals)
            return blk, band, g

        queue = []
        S = jnp.int32(0)
        for p in range(QD):
            lp, sp, ap = ginfo_fire(jnp.int32(p), S, jnp.int32(1))
            queue.extend((lp, sp, ap))
            S = ap

        def body(it, carry, out_hbm=out_hbm):
            D = carry[0]
            q = list(carry[1:])
            itF = jnp.minimum(it + QD, jnp.int32(NIT - 1))
            active = (it + QD < NIT).astype(jnp.int32)
            lF, sF, aF = ginfo_fire(itF, q[-1], active)
            lA, sA, aA = q[0], q[1], q[2]
            drain(aA - D)
            blk, band, g = extract(it, lA, sA)

            @pl.when(jnp.logical_and(band == NBAND - 1, g == NGRP - 1))
            def _():
                pltpu.sync_copy(rbuf,
                                out_hbm.at[pl.ds(base + blk * BLK, BLK)])
            return tuple([aA] + q[3:] + [lF, sF, aF])

        lax.fori_loop(0, NIT, body, tuple([jnp.int32(0)] + queue))


def _dot_body(invu_hbm, invi_hbm, gu_hbm, gi_hbm, out_hbm,
              idx_u, idx_i, rows_u, rows_i, out_v, sem_u, sem_i):
    wid = lax.axis_index("s") * NUM_CORES + lax.axis_index("c")
    lane_ids = lax.iota(jnp.int32, LANES)
    pltpu.sync_copy(invu_hbm.at[pl.ds(wid * NBLK, NBLK)], idx_u)
    pltpu.sync_copy(invi_hbm.at[pl.ds(wid * NBLK, NBLK)], idx_i)

    def fire(k):
        s = k % 2
        cu = pltpu.async_copy(gu_hbm.at[idx_u.at[k]], rows_u.at[s], sem_u)
        ci = pltpu.async_copy(gi_hbm.at[idx_i.at[k]], rows_i.at[s], sem_i)
        return cu, ci

    copies = [fire(0), fire(1)]
    for k in range(NBLK):
        s = k % 2
        cu, ci = copies[k]
        cu.wait()
        ci.wait()

        def block_body(b, carry, s=s, k=k):
            out_acc = jnp.zeros((LANES,), jnp.float32)
            for rr in range(LANES):
                r = b * LANES + rr
                acc = (rows_u[s, r, pl.ds(0, LANES)] *
                       rows_i[s, r, pl.ds(0, LANES)])
                for q in range(1, DIM // LANES):
                    acc = acc + (rows_u[s, r, pl.ds(q * LANES, LANES)] *
                                 rows_i[s, r, pl.ds(q * LANES, LANES)])
                out_acc = jnp.where(lane_ids == rr, jnp.sum(acc), out_acc)
            out_v[pl.ds(k * BLK + b * LANES, LANES)] = (
                1.0 / (1.0 + jnp.exp(-out_acc)))
            return carry

        lax.fori_loop(0, BLK // LANES, block_body, 0)
        if k + 2 < NBLK:
            copies.append(fire(k + 2))

    pltpu.sync_copy(out_v, out_hbm.at[pl.ds(wid * BPW, BPW)])


_extract = functools.partial(
    pl.kernel,
    mesh=plsc.VectorSubcoreMesh(core_axis_name="c", subcore_axis_name="s"),
    compiler_params=pltpu.CompilerParams(needs_layout_passes=False,
                                         use_tc_tiling_on_sc=True),
    out_type=(jax.ShapeDtypeStruct((BATCH, DIM), jnp.float32),
              jax.ShapeDtypeStruct((BATCH, DIM), jnp.float32)),
    scratch_types=[
        pltpu.VMEM((BPW,), jnp.int32),
        pltpu.VMEM((NSLOT, NBAND, 128), jnp.float32),
        pltpu.VMEM((BLK, DIM), jnp.float32),
        pltpu.SemaphoreType.DMA,
    ],
)(_extract_body)

_dot = functools.partial(
    pl.kernel,
    mesh=plsc.VectorSubcoreMesh(core_axis_name="c", subcore_axis_name="s"),
    compiler_params=pltpu.CompilerParams(needs_layout_passes=False,
                                         use_tc_tiling_on_sc=False),
    out_type=jax.ShapeDtypeStruct((BATCH,), jnp.float32),
    scratch_types=[
        pltpu.VMEM((NBLK, BLK), jnp.int32),
        pltpu.VMEM((NBLK, BLK), jnp.int32),
        pltpu.VMEM((2, BLK, DIM), jnp.float32),
        pltpu.VMEM((2, BLK, DIM), jnp.float32),
        pltpu.VMEM((BPW,), jnp.float32),
        pltpu.SemaphoreType.DMA,
        pltpu.SemaphoreType.DMA,
    ],
)(_dot_body)


def kernel(users, items, embedding_user, embedding_item):
    u = users.astype(jnp.int32)
    it = items.astype(jnp.int32)
    eu_t = embedding_user.T
    ei_t = embedding_item.T
    ou = jnp.argsort(u).astype(jnp.int32)
    oi = jnp.argsort(it).astype(jnp.int32)
    su = jnp.take(u, ou)
    si = jnp.take(it, oi)
    ar = jnp.arange(BATCH, dtype=jnp.int32)
    inv_u = jnp.zeros((BATCH,), jnp.int32).at[ou].set(ar)
    inv_i = jnp.zeros((BATCH,), jnp.int32).at[oi].set(ar)
    gu, gi = _extract(su, si, eu_t, ei_t)
    return _dot(inv_u.reshape(NW * NBLK, BLK), inv_i.reshape(NW * NBLK, BLK),
                gu, gi)
